# baseline (speedup 1.0000x reference)
import functools
import jax
import jax.numpy as jnp
from jax import lax
from jax.experimental import pallas as pl
from jax.experimental.pallas import tpu as pltpu
from jax.experimental.pallas import tpu_sc as plsc

_CH = 1024


def _sc_body(x_hbm, w_hbm, b_hbm, o_hbm, xbuf, obuf, wbuf, bbuf):
    pltpu.sync_copy(w_hbm, wbuf)
    pltpu.sync_copy(b_hbm, bbuf)
    cid = lax.axis_index("c")
    sid = lax.axis_index("s")
    wid = sid * 2 + cid
    bidx = wid // 4
    base = (wid % 4) * 65536

    bv = bbuf[pl.ds(0, 16)]
    bs = [bv[o] for o in range(16)]
    ws = []
    for o in range(16):
        w0 = wbuf[o, pl.ds(0, 16)]
        w1 = wbuf[o, pl.ds(16, 16)]
        ws.append([w0[c] for c in range(16)] + [w1[c] for c in range(16)])

    def chunk(i, carry):
        n0 = base + i * _CH
        pltpu.sync_copy(x_hbm.at[bidx, :, pl.ds(n0, _CH)], xbuf)

        def grp(g, c2):
            s = pl.ds(g * 16, 16)
            xs = [xbuf[c, s] for c in range(32)]
            for o in range(16):
                acc = jnp.full((16,), bs[o], jnp.float32)
                for c in range(32):
                    acc = acc + ws[o][c] * xs[c]
                obuf[o, s] = acc
            return c2

        lax.fori_loop(0, _CH // 16, grp, 0)
        pltpu.sync_copy(obuf, o_hbm.at[bidx, :, pl.ds(n0, _CH)])
        return carry

    lax.fori_loop(0, 64, chunk, 0)


def kernel(x, W, b):
    B, C, H, Wd = x.shape
    O = W.shape[0]
    N = H * Wd
    xf = x.reshape(B, C, N)
    mesh = plsc.VectorSubcoreMesh(core_axis_name="c", subcore_axis_name="s")
    run = functools.partial(
        pl.kernel,
        mesh=mesh,
        out_type=jax.ShapeDtypeStruct((B, O, N), jnp.float32),
        scratch_types=[
            pltpu.VMEM((C, _CH), jnp.float32),
            pltpu.VMEM((O, _CH), jnp.float32),
            pltpu.VMEM((O, C), jnp.float32),
            pltpu.VMEM((O,), jnp.float32),
        ],
    )(_sc_body)
    out = run(xf, W, b)
    return out.reshape(B, O, H, Wd)
